# baseline (speedup 1.0000x reference)
import jax
import jax.numpy as jnp
from jax import lax
from jax.experimental import pallas as pl
from jax.experimental.pallas import tpu as pltpu
from jax.experimental.pallas import tpu_sc as plsc

VOCAB = 100000
DIM = 64
BATCH = 16384

NUM_CORES = 2
NUM_SUBCORES = 16
LANES = 16
NW = NUM_CORES * NUM_SUBCORES
BPW = BATCH // NW
GROUPS = BPW // LANES


def _glove_body(tix_hbm, cix_hbm, temb_hbm, tb_hbm, cemb_hbm, cb_hbm,
                out_hbm, idx_t, idx_c, idx_th, idx_ch, te, ce,
                tbr, cbr, outv, sem0, sem1, sem2, sem3):
    wid = lax.axis_index("s") * NUM_CORES + lax.axis_index("c")
    base = wid * BPW

    pltpu.sync_copy(tix_hbm.at[pl.ds(base, BPW)], idx_t)
    pltpu.sync_copy(cix_hbm.at[pl.ds(base, BPW)], idx_c)

    def shift_body(g, carry):
        j0 = g * LANES
        idx_th[pl.ds(j0, LANES)] = lax.shift_right_logical(idx_t[pl.ds(j0, LANES)], 4)
        idx_ch[pl.ds(j0, LANES)] = lax.shift_right_logical(idx_c[pl.ds(j0, LANES)], 4)
        return carry

    lax.fori_loop(0, GROUPS, shift_body, 0)

    g0 = pltpu.async_copy(temb_hbm.at[idx_t], te, sem0)
    g1 = pltpu.async_copy(cemb_hbm.at[idx_c], ce, sem1)
    g2 = pltpu.async_copy(tb_hbm.at[idx_th], tbr, sem2)
    g3 = pltpu.async_copy(cb_hbm.at[idx_ch], cbr, sem3)
    g0.wait()
    g1.wait()
    g2.wait()
    g3.wait()

    lane = lax.iota(jnp.int32, LANES)
    dn = lax.GatherDimensionNumbers(
        offset_dims=(), collapsed_slice_dims=(0,), start_index_map=(0,))
    perms = [(lane ^ sh).reshape(LANES, 1) for sh in (1, 2, 4, 8)]

    def hsum(v):
        for p_ix in perms:
            v = v + lax.gather(v, p_ix, dn, slice_sizes=(1,),
                               mode=lax.GatherScatterMode.PROMISE_IN_BOUNDS)
        return v

    def group(g, carry):
        j0 = g * LANES
        row16 = j0 + lane
        tb_sel = plsc.load_gather(tbr, [row16, idx_t[pl.ds(j0, LANES)] & 15])
        cb_sel = plsc.load_gather(cbr, [row16, idx_c[pl.ds(j0, LANES)] & 15])
        acc = tb_sel + cb_sel
        for r in range(LANES):
            row = j0 + r
            p = te[row, pl.ds(0, LANES)] * ce[row, pl.ds(0, LANES)]
            for k in range(1, DIM // LANES):
                p = p + te[row, pl.ds(k * LANES, LANES)] * ce[row, pl.ds(k * LANES, LANES)]
            acc = jnp.where(lane == r, hsum(p) + acc, acc)
        outv[pl.ds(j0, LANES)] = acc
        return carry

    lax.fori_loop(0, GROUPS, group, 0)

    pltpu.sync_copy(outv, out_hbm.at[pl.ds(base, BPW)])


@jax.jit
def kernel(inputs, target_emb, target_bias, context_emb, context_bias):
    t_ix = inputs[:, 0].astype(jnp.int32)
    c_ix = inputs[:, 1].astype(jnp.int32)
    tb = target_bias.reshape(VOCAB // LANES, LANES)
    cb = context_bias.reshape(VOCAB // LANES, LANES)

    mesh = plsc.VectorSubcoreMesh(
        core_axis_name="c", subcore_axis_name="s",
        num_cores=NUM_CORES, num_subcores=NUM_SUBCORES)

    run = pl.kernel(
        _glove_body,
        out_type=jax.ShapeDtypeStruct((BATCH,), jnp.float32),
        mesh=mesh,
        compiler_params=pltpu.CompilerParams(
            use_tc_tiling_on_sc=False, needs_layout_passes=False),
        scratch_types=[
            pltpu.VMEM((BPW,), jnp.int32),
            pltpu.VMEM((BPW,), jnp.int32),
            pltpu.VMEM((BPW,), jnp.int32),
            pltpu.VMEM((BPW,), jnp.int32),
            pltpu.VMEM((BPW, DIM), jnp.float32),
            pltpu.VMEM((BPW, DIM), jnp.float32),
            pltpu.VMEM((BPW, LANES), jnp.float32),
            pltpu.VMEM((BPW, LANES), jnp.float32),
            pltpu.VMEM((BPW,), jnp.float32),
            pltpu.SemaphoreType.DMA,
            pltpu.SemaphoreType.DMA,
            pltpu.SemaphoreType.DMA,
            pltpu.SemaphoreType.DMA,
        ],
    )
    out = run(t_ix, c_ix, target_emb, tb, context_emb, cb)
    return out.reshape(BATCH, 1)
